# baseline (speedup 1.0000x reference)
import functools

import jax
import jax.numpy as jnp
from jax import lax
from jax.experimental import pallas as pl
from jax.experimental.pallas import tpu as pltpu
from jax.experimental.pallas import tpu_sc as plsc

B, C, N = 8, 64, 16384
P, S = 2048, 32
NW = 32
WPB = NW // B
CPW = C // WPB
CBLK = 4
NSWEEP = CPW // CBLK
PCH = 128
NCH = P // PCH
T = NSWEEP * NCH
NBLK = T // 2
BPS = NCH // 2

_mesh = plsc.VectorSubcoreMesh(core_axis_name="c", subcore_axis_name="s")


@functools.partial(
    pl.kernel,
    mesh=_mesh,
    out_type=jax.ShapeDtypeStruct((B, C, S, P), jnp.float32),
    scratch_types=[
        pltpu.VMEM((CBLK, N), jnp.float32),
        pltpu.VMEM((2, S, PCH), jnp.int32),
        pltpu.VMEM((2, CBLK, S, PCH), jnp.float32),
        pltpu.SemaphoreType.DMA((2,)),
        pltpu.SemaphoreType.DMA((2,)),
        pltpu.SemaphoreType.DMA,
    ],
    compiler_params=pltpu.CompilerParams(needs_layout_passes=False),
)
def _group_sc(feat_hbm, idx_hbm, out_hbm, feat_v, idx_v, out_v,
              isem, osem, fsem):
    cid = lax.axis_index("c")
    sid = lax.axis_index("s")
    w = sid * 2 + cid
    b = w // WPB
    c0 = (w % WPB) * CPW

    def idx_copy(blk, buf):
        h0 = lax.rem(blk, BPS) * PCH
        return pltpu.make_async_copy(
            idx_hbm.at[b, :, pl.ds(h0, PCH)], idx_v.at[buf], isem.at[buf])

    def out_copy(t, buf):
        cbase = c0 + (t // NCH) * CBLK
        p0 = lax.rem(t, NCH) * PCH
        return pltpu.make_async_copy(
            out_v.at[buf],
            out_hbm.at[b, pl.ds(cbase, CBLK), :, pl.ds(p0, PCH)],
            osem.at[buf])

    def feat_copy(sweep):
        cbase = c0 + sweep * CBLK
        return pltpu.make_async_copy(
            feat_hbm.at[b, pl.ds(cbase, CBLK), :], feat_v, fsem)

    ccv = [jnp.full((16,), cc, jnp.int32) for cc in range(CBLK)]

    def do_chunk(t, ibuf, hoff, obuf, guard, feat_boundary, qp):
        if guard:
            @pl.when(t > 1)
            def _():
                out_copy(t - 2, obuf).wait()
        else:
            out_copy(t - 2, obuf).wait()
        if feat_boundary:
            @pl.when(lax.rem(qp, BPS // 2) == 0)
            def _():
                feat_copy(0).wait()

        @plsc.parallel_loop(0, (PCH // 32) * S, unroll=4)
        def _gather(i):
            pg = lax.shift_right_logical(i, 5)
            s = lax.bitwise_and(i, S - 1)
            pbase = pg * 32
            ivp = idx_v[ibuf, s, pl.ds(hoff + pg * 16, 16)]
            iva = lax.bitwise_and(ivp, 0xFFFF)
            ivb = lax.shift_right_logical(ivp, 16)
            for cc in range(CBLK):
                out_v[obuf, cc, s, pl.ds(pbase, 16)] = plsc.load_gather(
                    feat_v, [ccv[cc], iva])
                out_v[obuf, cc, s, pl.ds(pbase + 16, 16)] = (
                    plsc.load_gather(feat_v, [ccv[cc], ivb]))

        out_copy(t, obuf).start()

    idx_copy(0, 0).start()

    def quad(qp, _):
        blk_a = 2 * qp
        t0 = 4 * qp

        @pl.when(lax.rem(qp, BPS // 2) == 0)
        def _():
            feat_copy(qp // (BPS // 2)).start()

        idx_copy(blk_a, 0).wait()
        idx_copy(blk_a + 1, 1).start()
        do_chunk(t0, 0, 0, 0, True, True, qp)
        do_chunk(t0 + 1, 0, PCH // 2, 1, True, False, qp)

        idx_copy(blk_a + 1, 1).wait()
        @pl.when(blk_a + 2 < NBLK)
        def _():
            idx_copy(blk_a + 2, 0).start()
        do_chunk(t0 + 2, 1, 0, 0, False, False, qp)
        do_chunk(t0 + 3, 1, PCH // 2, 1, False, False, qp)
        return 0

    lax.fori_loop(0, T // 4, quad, 0)

    out_copy(T - 2, 0).wait()
    out_copy(T - 1, 1).wait()


def kernel(features, idx):
    idx_t = jnp.transpose(idx.astype(jnp.int32), (0, 2, 1))
    v = idx_t.reshape(B, S, P // 32, 2, 16)
    packed = jnp.bitwise_or(v[..., 0, :],
                            jnp.left_shift(v[..., 1, :], 16))
    packed = packed.reshape(B, S, P // 2)
    out = _group_sc(features, packed)
    return jnp.transpose(out, (0, 1, 3, 2))


# speedup vs baseline: 1.0955x; 1.0955x over previous
import functools

import jax
import jax.numpy as jnp
from jax import lax
from jax.experimental import pallas as pl
from jax.experimental.pallas import tpu as pltpu
from jax.experimental.pallas import tpu_sc as plsc

B, C, N = 8, 64, 16384
P, S = 2048, 32
NW = 32
WPB = NW // B
CPW = C // WPB
CBLK = 4
NSWEEP = CPW // CBLK
PCH = 128
NCH = P // PCH
T = NSWEEP * NCH
PPS = NCH // 2

_mesh = plsc.VectorSubcoreMesh(core_axis_name="c", subcore_axis_name="s")


@functools.partial(
    pl.kernel,
    mesh=_mesh,
    out_type=jax.ShapeDtypeStruct((B, C, S, P), jnp.float32),
    scratch_types=[
        pltpu.VMEM((CBLK, N), jnp.float32),
        pltpu.VMEM((2, S, PCH), jnp.int32),
        pltpu.VMEM((2, CBLK, S, PCH), jnp.float32),
        pltpu.SemaphoreType.DMA((2,)),
        pltpu.SemaphoreType.DMA((2,)),
        pltpu.SemaphoreType.DMA,
    ],
    compiler_params=pltpu.CompilerParams(needs_layout_passes=False),
)
def _group_sc(feat_hbm, idx_hbm, out_hbm, feat_v, idx_v, out_v,
              isem, osem, fsem):
    cid = lax.axis_index("c")
    sid = lax.axis_index("s")
    w = sid * 2 + cid
    b = w // WPB
    c0 = (w % WPB) * CPW

    def idx_copy(t, buf):
        p0 = lax.rem(t, NCH) * PCH
        return pltpu.make_async_copy(
            idx_hbm.at[b, :, pl.ds(p0, PCH)], idx_v.at[buf], isem.at[buf])

    def out_copy(t, buf):
        cbase = c0 + (t // NCH) * CBLK
        p0 = lax.rem(t, NCH) * PCH
        return pltpu.make_async_copy(
            out_v.at[buf],
            out_hbm.at[b, pl.ds(cbase, CBLK), :, pl.ds(p0, PCH)],
            osem.at[buf])

    def feat_copy(sweep):
        cbase = c0 + sweep * CBLK
        return pltpu.make_async_copy(
            feat_hbm.at[b, pl.ds(cbase, CBLK), :], feat_v, fsem)

    ccv = [jnp.full((16,), cc, jnp.int32) for cc in range(CBLK)]

    def do_chunk(tp, t, buf, feat_boundary):
        idx_copy(t, buf).wait()
        @pl.when(t + 1 < T)
        def _():
            idx_copy(t + 1, 1 - buf).start()
        @pl.when(tp > 0)
        def _():
            out_copy(t - 2, buf).wait()
        if feat_boundary:
            @pl.when(lax.rem(tp, PPS) == 0)
            def _():
                feat_copy(0).wait()

        @plsc.parallel_loop(0, (PCH // 16) * S, unroll=8)
        def _gather(i):
            pg = lax.shift_right_logical(i, 5)
            s = lax.bitwise_and(i, S - 1)
            pbase = pg * 16
            iv = idx_v[buf, s, pl.ds(pbase, 16)]
            for cc in range(CBLK):
                out_v[buf, cc, s, pl.ds(pbase, 16)] = plsc.load_gather(
                    feat_v, [ccv[cc], iv])

        out_copy(t, buf).start()

    idx_copy(0, 0).start()

    def pair(tp, _):
        @pl.when(lax.rem(tp, PPS) == 0)
        def _():
            feat_copy(tp // PPS).start()

        do_chunk(tp, 2 * tp, 0, True)
        do_chunk(tp, 2 * tp + 1, 1, False)
        return 0

    lax.fori_loop(0, T // 2, pair, 0)

    out_copy(T - 2, 0).wait()
    out_copy(T - 1, 1).wait()


def kernel(features, idx):
    idx_t = jnp.transpose(idx.astype(jnp.int32), (0, 2, 1))
    out = _group_sc(features, idx_t)
    return jnp.transpose(out, (0, 1, 3, 2))
